# baseline (speedup 1.0000x reference)
import functools

import jax
import jax.numpy as jnp
from jax import lax
from jax.experimental import pallas as pl
from jax.experimental.pallas import tpu as pltpu
from jax.experimental.pallas import tpu_sc as plsc

NC, NS = 2, 16
NW = NC * NS
CHUNK = 32

EPS = 1e-12


def _sc_gather(weight, flat_ids):
    n_tok = flat_ids.shape[0]
    _, h = weight.shape
    b_per_w = n_tok // NW
    n_chunks = b_per_w // CHUNK
    mesh = plsc.VectorSubcoreMesh(core_axis_name="c", subcore_axis_name="s")

    @functools.partial(
        pl.kernel,
        out_type=jax.ShapeDtypeStruct((n_tok, h), jnp.float32),
        mesh=mesh,
        scratch_types=[
            pltpu.VMEM((n_chunks, CHUNK), jnp.int32),
            pltpu.VMEM((2, CHUNK, h), jnp.float32),
            pltpu.SemaphoreType.DMA,
            pltpu.SemaphoreType.DMA,
        ],
    )
    def gather_kernel(weight_hbm, ids_hbm, out_hbm, idx_v, rows_v, gsem, osem):
        wid = lax.axis_index("s") * NC + lax.axis_index("c")
        base = wid * b_per_w
        for c in range(n_chunks):
            pltpu.sync_copy(ids_hbm.at[pl.ds(base + c * CHUNK, CHUNK)], idx_v.at[c])

        gathers = [None] * n_chunks
        writes = [None] * n_chunks

        def start_gather(c):
            gathers[c] = pltpu.async_copy(
                weight_hbm.at[idx_v.at[c]], rows_v.at[c % 2], gsem)

        start_gather(0)
        if n_chunks > 1:
            start_gather(1)
        for c in range(n_chunks):
            gathers[c].wait()
            writes[c] = pltpu.async_copy(
                rows_v.at[c % 2], out_hbm.at[pl.ds(base + c * CHUNK, CHUNK)], osem)
            nxt = c + 2
            if nxt < n_chunks:
                writes[c].wait()
                start_gather(nxt)
        for c in range(max(0, n_chunks - 2), n_chunks):
            writes[c].wait()

    return gather_kernel(weight, flat_ids)


def _tc_epilogue(x, pos, tt_table, ttid_f, gamma, beta, batch, seq):
    h = x.shape[-1]

    def body(x_ref, pos_ref, tt_ref, id_ref, g_ref, b_ref, o_ref):
        ids = id_ref[0, 0, :].reshape(seq, 1)
        v = x_ref[...] + pos_ref[...] + tt_ref[0] + ids * (tt_ref[1] - tt_ref[0])
        ones = jnp.ones((h, 1), jnp.float32)
        s1 = jax.lax.dot(v, ones) * (1.0 / h)
        s2 = jax.lax.dot(v * v, ones) * (1.0 / h)
        var = s2 - s1 * s1
        o_ref[...] = ((v - s1) * lax.rsqrt(var + EPS)) * g_ref[...] + b_ref[...]

    return pl.pallas_call(
        body,
        grid=(batch,),
        in_specs=[
            pl.BlockSpec((seq, h), lambda b: (b, 0)),
            pl.BlockSpec((seq, h), lambda b: (0, 0)),
            pl.BlockSpec((2, h), lambda b: (0, 0)),
            pl.BlockSpec((1, 1, seq), lambda b: (b, 0, 0)),
            pl.BlockSpec((1, h), lambda b: (0, 0)),
            pl.BlockSpec((1, h), lambda b: (0, 0)),
        ],
        out_specs=pl.BlockSpec((seq, h), lambda b: (b, 0)),
        out_shape=jax.ShapeDtypeStruct((batch * seq, h), jnp.float32),
    )(x, pos, tt_table, ttid_f, gamma, beta)


N_SLICES = 2


def kernel(input_ids, token_type_ids, weight, token_type_embeddings,
           position_embeddings, ln_gamma, ln_beta):
    batch, seq = input_ids.shape
    h = weight.shape[-1]
    sb = batch // N_SLICES
    ids = input_ids.astype(jnp.int32)
    ttid_f = token_type_ids.reshape(batch, 1, seq).astype(jnp.float32)
    outs = []
    for s in range(N_SLICES):
        flat_ids = ids[s * sb:(s + 1) * sb].reshape(-1)
        gathered = _sc_gather(weight, flat_ids)
        outs.append(_tc_epilogue(
            gathered, position_embeddings, token_type_embeddings,
            ttid_f[s * sb:(s + 1) * sb], ln_gamma.reshape(1, h),
            ln_beta.reshape(1, h), sb, seq))
    return jnp.concatenate(outs, axis=0).reshape(batch, seq, h)


# speedup vs baseline: 1.2635x; 1.2635x over previous
import functools

import jax
import jax.numpy as jnp
from jax import lax
from jax.experimental import pallas as pl
from jax.experimental.pallas import tpu as pltpu
from jax.experimental.pallas import tpu_sc as plsc

NC, NS = 2, 16
NW = NC * NS
CHUNK = 64

EPS = 1e-12


def _sc_gather(weight, ids):
    batch, seq = ids.shape
    n_tok = batch * seq
    _, h = weight.shape
    b_per_w = n_tok // NW
    n_chunks = b_per_w // CHUNK
    w_per_row = seq // b_per_w
    mesh = plsc.VectorSubcoreMesh(core_axis_name="c", subcore_axis_name="s")

    @functools.partial(
        pl.kernel,
        out_type=jax.ShapeDtypeStruct((n_tok, h), jnp.float32),
        mesh=mesh,
        scratch_types=[
            pltpu.VMEM((n_chunks, CHUNK), jnp.int32),
            pltpu.VMEM((2, CHUNK, h), jnp.float32),
            pltpu.SemaphoreType.DMA,
            pltpu.SemaphoreType.DMA,
        ],
    )
    def gather_kernel(weight_hbm, ids_hbm, out_hbm, idx_v, rows_v, gsem, osem):
        wid = lax.axis_index("s") * NC + lax.axis_index("c")
        base = wid * b_per_w
        row = wid // w_per_row
        col0 = (wid % w_per_row) * b_per_w
        for c in range(n_chunks):
            pltpu.sync_copy(ids_hbm.at[row, pl.ds(col0 + c * CHUNK, CHUNK)],
                            idx_v.at[c])

        gathers = [None] * n_chunks
        writes = [None] * n_chunks

        def start_gather(c):
            gathers[c] = pltpu.async_copy(
                weight_hbm.at[idx_v.at[c]], rows_v.at[c % 2], gsem)

        start_gather(0)
        if n_chunks > 1:
            start_gather(1)
        for c in range(n_chunks):
            gathers[c].wait()
            writes[c] = pltpu.async_copy(
                rows_v.at[c % 2], out_hbm.at[pl.ds(base + c * CHUNK, CHUNK)], osem)
            nxt = c + 2
            if nxt < n_chunks:
                writes[c].wait()
                start_gather(nxt)
        for c in range(max(0, n_chunks - 2), n_chunks):
            writes[c].wait()

    return gather_kernel(weight, ids)


def _tc_epilogue(x, pos, tt_table, ttids, gamma, beta, batch, seq):
    h = x.shape[-1]

    def body(x_ref, pos_ref, tt_ref, id_ref, g_ref, b_ref, o_ref):
        ids = id_ref[0, 0, :].astype(jnp.float32).reshape(seq, 1)
        v = x_ref[...] + pos_ref[...] + tt_ref[0] + ids * (tt_ref[1] - tt_ref[0])
        ones = jnp.ones((h, 1), jnp.float32)
        s1 = jax.lax.dot(v, ones) * (1.0 / h)
        s2 = jax.lax.dot(v * v, ones) * (1.0 / h)
        var = s2 - s1 * s1
        o_ref[0] = ((v - s1) * lax.rsqrt(var + EPS)) * g_ref[...] + b_ref[...]

    return pl.pallas_call(
        body,
        grid=(batch,),
        in_specs=[
            pl.BlockSpec((seq, h), lambda b: (b, 0)),
            pl.BlockSpec((seq, h), lambda b: (0, 0)),
            pl.BlockSpec((2, h), lambda b: (0, 0)),
            pl.BlockSpec((1, 1, seq), lambda b: (b, 0, 0)),
            pl.BlockSpec((1, h), lambda b: (0, 0)),
            pl.BlockSpec((1, h), lambda b: (0, 0)),
        ],
        out_specs=pl.BlockSpec((1, seq, h), lambda b: (b, 0, 0)),
        out_shape=jax.ShapeDtypeStruct((batch, seq, h), jnp.float32),
    )(x, pos, tt_table, ttids, gamma, beta)


def kernel(input_ids, token_type_ids, weight, token_type_embeddings,
           position_embeddings, ln_gamma, ln_beta):
    batch, seq = input_ids.shape
    h = weight.shape[-1]
    gathered = _sc_gather(weight, input_ids)
    return _tc_epilogue(
        gathered, position_embeddings, token_type_embeddings,
        token_type_ids.reshape(batch, 1, seq), ln_gamma.reshape(1, h),
        ln_beta.reshape(1, h), batch, seq)


# speedup vs baseline: 1.2926x; 1.0230x over previous
import functools

import jax
import jax.numpy as jnp
from jax import lax
from jax.experimental import pallas as pl
from jax.experimental.pallas import tpu as pltpu
from jax.experimental.pallas import tpu_sc as plsc

NC, NS = 2, 16
NW = NC * NS
CHUNK = 64

EPS = 1e-12


def _sc_gather(weight, ids):
    batch, seq = ids.shape
    n_tok = batch * seq
    _, h = weight.shape
    b_per_w = n_tok // NW
    n_chunks = b_per_w // CHUNK
    w_per_row = seq // b_per_w
    mesh = plsc.VectorSubcoreMesh(core_axis_name="c", subcore_axis_name="s")

    @functools.partial(
        pl.kernel,
        out_type=jax.ShapeDtypeStruct((n_tok, h), jnp.float32),
        mesh=mesh,
        scratch_types=[
            pltpu.VMEM((b_per_w,), jnp.int32),
            pltpu.VMEM((2, CHUNK, h), jnp.float32),
            pltpu.SemaphoreType.DMA,
            pltpu.SemaphoreType.DMA,
        ],
    )
    def gather_kernel(weight_hbm, ids_hbm, out_hbm, idx_v, rows_v, gsem, osem):
        wid = lax.axis_index("s") * NC + lax.axis_index("c")
        base = wid * b_per_w
        row = wid // w_per_row
        col0 = (wid % w_per_row) * b_per_w
        pltpu.sync_copy(ids_hbm.at[row, pl.ds(col0, b_per_w)], idx_v)

        gathers = [None] * n_chunks
        writes = [None] * n_chunks

        def start_gather(c):
            gathers[c] = pltpu.async_copy(
                weight_hbm.at[idx_v.at[pl.ds(c * CHUNK, CHUNK)]],
                rows_v.at[c % 2], gsem)

        start_gather(0)
        if n_chunks > 1:
            start_gather(1)
        for c in range(n_chunks):
            gathers[c].wait()
            writes[c] = pltpu.async_copy(
                rows_v.at[c % 2], out_hbm.at[pl.ds(base + c * CHUNK, CHUNK)], osem)
            nxt = c + 2
            if nxt < n_chunks:
                writes[c].wait()
                start_gather(nxt)
        for c in range(max(0, n_chunks - 2), n_chunks):
            writes[c].wait()

    return gather_kernel(weight, ids)


def _tc_epilogue(x, pos, tt_table, ttids, gamma, beta, batch, seq):
    h = x.shape[-1]

    def body(x_ref, pos_ref, tt_ref, id_ref, g_ref, b_ref, o_ref):
        ids = id_ref[0, 0, :].astype(jnp.float32).reshape(seq, 1)
        v = x_ref[...] + pos_ref[...] + tt_ref[0] + ids * (tt_ref[1] - tt_ref[0])
        ones = jnp.ones((h, 1), jnp.float32)
        s1 = jax.lax.dot(v, ones) * (1.0 / h)
        s2 = jax.lax.dot(v * v, ones) * (1.0 / h)
        var = s2 - s1 * s1
        o_ref[0] = ((v - s1) * lax.rsqrt(var + EPS)) * g_ref[...] + b_ref[...]

    return pl.pallas_call(
        body,
        grid=(batch,),
        in_specs=[
            pl.BlockSpec((seq, h), lambda b: (b, 0)),
            pl.BlockSpec((seq, h), lambda b: (0, 0)),
            pl.BlockSpec((2, h), lambda b: (0, 0)),
            pl.BlockSpec((1, 1, seq), lambda b: (b, 0, 0)),
            pl.BlockSpec((1, h), lambda b: (0, 0)),
            pl.BlockSpec((1, h), lambda b: (0, 0)),
        ],
        out_specs=pl.BlockSpec((1, seq, h), lambda b: (b, 0, 0)),
        out_shape=jax.ShapeDtypeStruct((batch, seq, h), jnp.float32),
    )(x, pos, tt_table, ttids, gamma, beta)


def kernel(input_ids, token_type_ids, weight, token_type_embeddings,
           position_embeddings, ln_gamma, ln_beta):
    batch, seq = input_ids.shape
    h = weight.shape[-1]
    gathered = _sc_gather(weight, input_ids)
    return _tc_epilogue(
        gathered, position_embeddings, token_type_embeddings,
        token_type_ids.reshape(batch, 1, seq), ln_gamma.reshape(1, h),
        ln_beta.reshape(1, h), batch, seq)
